# baseline (speedup 1.0000x reference)
import jax
import jax.numpy as jnp
from jax.experimental import pallas as pl

SEQ_BLOCK = 1024
BATCH_BLOCK = 2


def _add_block(x_ref, pos_ref, o_ref):
    o_ref[...] = x_ref[...] + pos_ref[...]


def kernel(x, pos_table):
    batch, seq_len, embed_dim = x.shape
    pos = pos_table[:seq_len]
    return pl.pallas_call(
        _add_block,
        grid=(seq_len // SEQ_BLOCK, batch // BATCH_BLOCK),
        in_specs=[
            pl.BlockSpec((BATCH_BLOCK, SEQ_BLOCK, embed_dim), lambda i, j: (j, i, 0)),
            pl.BlockSpec((SEQ_BLOCK, embed_dim), lambda i, j: (i, 0)),
        ],
        out_specs=pl.BlockSpec(
            (BATCH_BLOCK, SEQ_BLOCK, embed_dim), lambda i, j: (j, i, 0)
        ),
        out_shape=jax.ShapeDtypeStruct((batch, seq_len, embed_dim), x.dtype),
    )(x, pos)
